# initial kernel scaffold (unmeasured)
import jax
import jax.numpy as jnp
from jax import lax
from jax.experimental import pallas as pl
from jax.experimental.pallas import tpu as pltpu

N_DEV = 4
BLK = 256


def kernel(x, w_mat):
    k_total, k_per = x.shape
    _, n = w_mat.shape
    m_per = k_total // N_DEV

    def body(x_ref, w_ref, out_ref, xb_ref, comm_ref, send_sems, recv_sems):
        my = lax.axis_index("i")

        xb_ref[...] = x_ref[...].astype(jnp.bfloat16)

        rdmas = []
        for d in range(1, N_DEV):
            j = (my + d) % N_DEV
            rdma = pltpu.make_async_remote_copy(
                src_ref=xb_ref.at[pl.ds(j * m_per, m_per), :],
                dst_ref=comm_ref.at[d - 1],
                send_sem=send_sems.at[d - 1],
                recv_sem=recv_sems.at[d - 1],
                device_id=(j,),
                device_id_type=pl.DeviceIdType.MESH,
            )
            rdma.start()
            rdmas.append(rdma)

        wb = w_ref[pl.ds(my * BLK, BLK), :].astype(jnp.bfloat16)
        out_ref[...] = jnp.dot(
            xb_ref[pl.ds(my * m_per, m_per), :], wb,
            preferred_element_type=jnp.float32,
        )

        for d in range(1, N_DEV):
            rdmas[d - 1].wait_recv()
            s = (my - d) % N_DEV
            wb = w_ref[pl.ds(s * BLK, BLK), :].astype(jnp.bfloat16)
            out_ref[...] += jnp.dot(
                comm_ref[d - 1], wb, preferred_element_type=jnp.float32
            )

        out_ref[...] = jnp.maximum(out_ref[...], 0.0)

        for d in range(1, N_DEV):
            rdmas[d - 1].wait_send()

    return pl.pallas_call(
        body,
        out_shape=jax.ShapeDtypeStruct((m_per, n), jnp.float32),
        in_specs=[
            pl.BlockSpec(memory_space=pltpu.VMEM),
            pl.BlockSpec(memory_space=pltpu.VMEM),
        ],
        out_specs=pl.BlockSpec(memory_space=pltpu.VMEM),
        scratch_shapes=[
            pltpu.VMEM((k_total, k_per), jnp.bfloat16),
            pltpu.VMEM((N_DEV - 1, m_per, k_per), jnp.bfloat16),
            pltpu.SemaphoreType.DMA((N_DEV - 1,)),
            pltpu.SemaphoreType.DMA((N_DEV - 1,)),
        ],
        compiler_params=pltpu.CompilerParams(collective_id=0),
    )(x, w_mat)


# baseline (device time: 14804 ns/iter reference)
import jax
import jax.numpy as jnp
from jax import lax
from jax.experimental import pallas as pl
from jax.experimental.pallas import tpu as pltpu

N_DEV = 4
BLK = 256


def kernel(x, w_mat):
    k_total, k_per = x.shape
    _, n = w_mat.shape
    m_per = k_total // N_DEV

    def body(x_ref, w_ref, out_ref, xb_ref, comm_ref, send_sems, recv_sems):
        my = lax.axis_index("i")

        xb_ref[...] = x_ref[...].astype(jnp.bfloat16)

        rdmas = []
        for d in range(1, N_DEV):
            j = (my + d) % N_DEV
            rdma = pltpu.make_async_remote_copy(
                src_ref=xb_ref.at[pl.ds(j * m_per, m_per), :],
                dst_ref=comm_ref.at[d - 1],
                send_sem=send_sems.at[d - 1],
                recv_sem=recv_sems.at[d - 1],
                device_id=(j,),
                device_id_type=pl.DeviceIdType.MESH,
            )
            rdma.start()
            rdmas.append(rdma)

        wb = w_ref[pl.ds(my * BLK, BLK), :].astype(jnp.bfloat16)
        out_ref[...] = jnp.dot(
            xb_ref[pl.ds(my * m_per, m_per), :], wb,
            preferred_element_type=jnp.float32,
        )

        for d in range(1, N_DEV):
            rdmas[d - 1].wait_recv()
            s = (my - d) % N_DEV
            wb = w_ref[pl.ds(s * BLK, BLK), :].astype(jnp.bfloat16)
            out_ref[...] += jnp.dot(
                comm_ref[d - 1], wb, preferred_element_type=jnp.float32
            )

        out_ref[...] = jnp.maximum(out_ref[...], 0.0)

        for d in range(1, N_DEV):
            rdmas[d - 1].wait_send()

    return pl.pallas_call(
        body,
        out_shape=jax.ShapeDtypeStruct((m_per, n), jnp.float32),
        in_specs=[
            pl.BlockSpec(memory_space=pltpu.VMEM),
            pl.BlockSpec(memory_space=pltpu.VMEM),
        ],
        out_specs=pl.BlockSpec(memory_space=pltpu.VMEM),
        scratch_shapes=[
            pltpu.VMEM((k_total, k_per), jnp.bfloat16),
            pltpu.VMEM((N_DEV - 1, m_per, k_per), jnp.bfloat16),
            pltpu.SemaphoreType.DMA((N_DEV - 1,)),
            pltpu.SemaphoreType.DMA((N_DEV - 1,)),
        ],
    )(x, w_mat)


# device time: 12583 ns/iter; 1.1765x vs baseline; 1.1765x over previous
import jax
import jax.numpy as jnp
from jax import lax
from jax.experimental import pallas as pl
from jax.experimental.pallas import tpu as pltpu

N_DEV = 4
BLK = 256
ORDER = (1, 3, 2)


def kernel(x, w_mat):
    k_total, k_per = x.shape
    _, n = w_mat.shape
    m_per = k_total // N_DEV

    def body(x_ref, w_hbm, out_ref, xb_ref, w_ref, comm_ref,
             send_sems, recv_sems, w_sem):
        my = lax.axis_index("i")

        barrier_sem = pltpu.get_barrier_semaphore()
        for d in range(1, N_DEV):
            pl.semaphore_signal(
                barrier_sem, inc=1,
                device_id=((my + d) % N_DEV,),
                device_id_type=pl.DeviceIdType.MESH,
            )

        wcopy = pltpu.make_async_copy(w_hbm, w_ref, w_sem)
        wcopy.start()

        xb_ref[...] = x_ref[...].astype(jnp.bfloat16)

        pl.semaphore_wait(barrier_sem, N_DEV - 1)

        rdmas = {}
        for d in ORDER:
            j = (my + d) % N_DEV
            rdma = pltpu.make_async_remote_copy(
                src_ref=xb_ref.at[pl.ds(j * m_per, m_per), :],
                dst_ref=comm_ref.at[d - 1],
                send_sem=send_sems.at[d - 1],
                recv_sem=recv_sems.at[d - 1],
                device_id=(j,),
                device_id_type=pl.DeviceIdType.MESH,
            )
            rdma.start()
            rdmas[d] = rdma

        wcopy.wait()
        wb = w_ref[pl.ds(my * BLK, BLK), :].astype(jnp.bfloat16)
        out_ref[...] = jnp.dot(
            xb_ref[pl.ds(my * m_per, m_per), :], wb,
            preferred_element_type=jnp.float32,
        )

        for d in ORDER:
            rdmas[d].wait_recv()
            s = (my - d) % N_DEV
            wb = w_ref[pl.ds(s * BLK, BLK), :].astype(jnp.bfloat16)
            acc = out_ref[...] + jnp.dot(
                comm_ref[d - 1], wb, preferred_element_type=jnp.float32
            )
            if d == ORDER[-1]:
                acc = jnp.maximum(acc, 0.0)
            out_ref[...] = acc

        for d in ORDER:
            rdmas[d].wait_send()

    return pl.pallas_call(
        body,
        out_shape=jax.ShapeDtypeStruct((m_per, n), jnp.float32),
        in_specs=[
            pl.BlockSpec(memory_space=pltpu.VMEM),
            pl.BlockSpec(memory_space=pl.ANY),
        ],
        out_specs=pl.BlockSpec(memory_space=pltpu.VMEM),
        scratch_shapes=[
            pltpu.VMEM((k_total, k_per), jnp.bfloat16),
            pltpu.VMEM((k_total, n), jnp.float32),
            pltpu.VMEM((N_DEV - 1, m_per, k_per), jnp.bfloat16),
            pltpu.SemaphoreType.DMA((N_DEV - 1,)),
            pltpu.SemaphoreType.DMA((N_DEV - 1,)),
            pltpu.SemaphoreType.DMA,
        ],
        compiler_params=pltpu.CompilerParams(collective_id=0),
    )(x, w_mat)
